# Initial kernel scaffold; baseline (speedup 1.0000x reference)
import jax
import jax.numpy as jnp
from jax.experimental import pallas as pl


def kernel(x, W_gate):
    raise NotImplementedError("write your pallas kernel here")


# speedup vs baseline: 1.6295x; 1.6295x over previous
import jax
import jax.numpy as jnp
from jax import lax
from jax.experimental import pallas as pl
from jax.experimental.pallas import tpu as pltpu

D_MODEL_ = 768
N_EXP_ = 64


def _router_body(x_ref, wt_ref, logits_ref, w_ref, i_ref):
    x_blk = x_ref[...]
    logits = jnp.dot(x_blk, wt_ref[...], preferred_element_type=jnp.float32)
    logits_ref[...] = logits

    bt = logits.shape[0]
    iota = lax.broadcasted_iota(jnp.int32, (bt, N_EXP_), 1)
    m1 = jnp.max(logits, axis=-1, keepdims=True)
    i1 = jnp.min(jnp.where(logits == m1, iota, N_EXP_), axis=-1, keepdims=True)
    masked = jnp.where(iota == i1, -jnp.inf, logits)
    m2 = jnp.max(masked, axis=-1, keepdims=True)
    i2 = jnp.min(jnp.where(masked == m2, iota, N_EXP_), axis=-1, keepdims=True)

    e = jnp.exp(m2 - m1)
    denom = 1.0 + e
    w1 = 1.0 / denom
    w2 = e / denom
    w_ref[...] = jnp.concatenate([w1, w2], axis=1)
    i_ref[...] = jnp.concatenate([i1, i2], axis=1)


def kernel(x, W_gate):
    B, S, D = x.shape
    T = B * S
    xf = x.reshape(T, D)
    wt = W_gate.T

    BT = 1024
    grid = (T // BT,)

    logits, weights, indices = pl.pallas_call(
        _router_body,
        grid=grid,
        in_specs=[
            pl.BlockSpec((BT, D), lambda i: (i, 0)),
            pl.BlockSpec((D, N_EXP_), lambda i: (0, 0)),
        ],
        out_specs=[
            pl.BlockSpec((BT, N_EXP_), lambda i: (i, 0)),
            pl.BlockSpec((BT, 2), lambda i: (i, 0)),
            pl.BlockSpec((BT, 2), lambda i: (i, 0)),
        ],
        out_shape=[
            jax.ShapeDtypeStruct((T, N_EXP_), jnp.float32),
            jax.ShapeDtypeStruct((T, 2), jnp.float32),
            jax.ShapeDtypeStruct((T, 2), jnp.int32),
        ],
    )(xf, wt)

    return (
        weights.reshape(B, S, 2),
        indices.reshape(B, S, 2),
        logits.reshape(B, S, N_EXP_),
    )


# speedup vs baseline: 1.8760x; 1.1513x over previous
import jax
import jax.numpy as jnp
from jax import lax
from jax.experimental import pallas as pl
from jax.experimental.pallas import tpu as pltpu

D_MODEL_ = 768
N_EXP_ = 64


def _router_body(x_ref, wt_ref, logits_ref, w_ref, i_ref):
    x_blk = x_ref[...]
    logits = jnp.dot(x_blk, wt_ref[...], preferred_element_type=jnp.float32)
    logits_ref[...] = logits

    bt = logits.shape[0]
    iota = lax.broadcasted_iota(jnp.int32, (bt, N_EXP_), 1)
    m1 = jnp.max(logits, axis=-1, keepdims=True)
    i1 = jnp.min(jnp.where(logits == m1, iota, N_EXP_), axis=-1, keepdims=True)
    masked = jnp.where(iota == i1, -jnp.inf, logits)
    m2 = jnp.max(masked, axis=-1, keepdims=True)
    i2 = jnp.min(jnp.where(masked == m2, iota, N_EXP_), axis=-1, keepdims=True)

    e = jnp.exp(m2 - m1)
    denom = 1.0 + e
    w1 = 1.0 / denom
    w2 = e / denom
    w_ref[...] = jnp.concatenate([w1, w2], axis=1)
    i_ref[...] = jnp.concatenate([i1, i2], axis=1)


def kernel(x, W_gate):
    B, S, D = x.shape
    T = B * S
    xf = x.reshape(T, D)
    wt = W_gate.T

    BT = 4096
    grid = (T // BT,)

    logits, weights, indices = pl.pallas_call(
        _router_body,
        grid=grid,
        in_specs=[
            pl.BlockSpec((BT, D), lambda i: (i, 0)),
            pl.BlockSpec((D, N_EXP_), lambda i: (0, 0)),
        ],
        out_specs=[
            pl.BlockSpec((BT, N_EXP_), lambda i: (i, 0)),
            pl.BlockSpec((BT, 2), lambda i: (i, 0)),
            pl.BlockSpec((BT, 2), lambda i: (i, 0)),
        ],
        out_shape=[
            jax.ShapeDtypeStruct((T, N_EXP_), jnp.float32),
            jax.ShapeDtypeStruct((T, 2), jnp.float32),
            jax.ShapeDtypeStruct((T, 2), jnp.int32),
        ],
    )(xf, wt)

    return (
        weights.reshape(B, S, 2),
        indices.reshape(B, S, 2),
        logits.reshape(B, S, N_EXP_),
    )
